# baseline (speedup 1.0000x reference)
import functools

import jax
import jax.numpy as jnp
from jax import lax
from jax.experimental import pallas as pl
from jax.experimental.pallas import tpu as pltpu
from jax.experimental.pallas import tpu_sc as plsc

_N = 10000
_E = 320000
_DIM = 64
_GAMMA = 1.0

_NC = 2
_NS = 16
_K = 128
_CH = 160
_R = 4
_EPT = _CH * _K
_EPAD = _NS * _EPT
_RPT = _N // _NS
_REC = 3 * _K

_BN = 2000


def _pre_body(f_ref, km_ref, kv_ref, out_ref):
    f = f_ref[...]
    dn = (((1,), (0,)), ((), ()))
    m = lax.dot_general(f[:, :_DIM], km_ref[...], dn,
                        precision=lax.Precision.HIGHEST,
                        preferred_element_type=jnp.float32)
    v = lax.dot_general(f[:, _DIM:], kv_ref[...], dn,
                        precision=lax.Precision.HIGHEST,
                        preferred_element_type=jnp.float32)
    m = jnp.where(m > 0.0, m, jnp.exp(m) - 1.0)
    v = jnp.maximum(v, 0.0)
    att = jnp.exp(-_GAMMA * v)
    out_ref[0] = m * att
    out_ref[1] = v * att * att


def _post_body(agg_ref, noise_ref, out_ref):
    out_ref[...] = agg_ref[0] + jnp.sqrt(agg_ref[1] + 1e-8) * noise_ref[...]


_sc_mesh = plsc.VectorSubcoreMesh(
    core_axis_name="c", subcore_axis_name="s", num_cores=_NC, num_subcores=_NS
)


@functools.partial(
    pl.kernel,
    out_type=jax.ShapeDtypeStruct((_NC, _NS, _RPT, _DIM), jnp.float32),
    mesh=_sc_mesh,
    compiler_params=pltpu.CompilerParams(
        needs_layout_passes=False, use_tc_tiling_on_sc=False),
    scratch_types=(
        [pltpu.VMEM((_R * _REC,), jnp.int32)]
        + [pltpu.VMEM((_R, _K), jnp.int32)]
        + [pltpu.VMEM((_K, _DIM), jnp.float32)] * _R
        + [pltpu.VMEM((_K, _DIM), jnp.float32)] * _R
        + [pltpu.VMEM_SHARED((_N, _DIM), jnp.float32)]
        + [pltpu.SemaphoreType.DMA] * (3 * _R)
    ),
)
def _edge_kernel(x2_hbm, edata_hbm, out_hbm,
                 idx_v, dstr_v,
                 g0, g1, g2, g3, s0, s1, s2, s3, agg_sh,
                 i0, i1, i2, i3, q0, q1, q2, q3, p0, p1, p2, p3):
    c = lax.axis_index("c")
    s = lax.axis_index("s")
    gbufs = (g0, g1, g2, g3)
    sbufs = (s0, s1, s2, s3)
    isems = (i0, i1, i2, i3)
    gsems = (q0, q1, q2, q3)
    ssems = (p0, p1, p2, p3)

    def zrow(e, carry):
        for q in range(_DIM // 16):
            g0[e, pl.ds(q * 16, 16)] = jnp.zeros((16,), jnp.float32)
        return carry

    lax.fori_loop(0, _K, zrow, 0)

    base = s * _RPT
    n_full = _RPT // _K
    rem = _RPT - n_full * _K

    def zcp(i, carry):
        pltpu.sync_copy(g0, agg_sh.at[pl.ds(base + i * _K, _K)])
        return carry

    lax.fori_loop(0, n_full, zcp, 0)
    if rem:
        pltpu.sync_copy(g0.at[pl.ds(0, rem)],
                        agg_sh.at[pl.ds(base + n_full * _K, rem)])
    plsc.subcore_barrier()

    def idx_copy(r, j):
        return pltpu.make_async_copy(
            edata_hbm.at[c, s, j], idx_v.at[pl.ds(r * _REC, _REC)], isems[r])

    def row_gather(r, j):
        return pltpu.make_async_copy(
            x2_hbm.at[idx_v.at[pl.ds(r * _REC, _K)]], gbufs[r], gsems[r])

    def row_scatter(r):
        return pltpu.async_copy(
            sbufs[r], agg_sh.at[dstr_v.at[r]], ssems[r], add=True)

    def row_scatter_wait(r):
        pltpu.make_async_copy(
            sbufs[r], agg_sh.at[dstr_v.at[r]], ssems[r]).wait()

    def scale(r, j):
        gbuf, sbuf = gbufs[r], sbufs[r]
        wbase = r * _REC + 2 * _K

        def group(g, c2):
            for e in range(16):
                eidx = g * 16 + e
                wb = plsc.bitcast(
                    plsc.load_gather(
                        idx_v, [jnp.full((16,), wbase + eidx, jnp.int32)]),
                    jnp.float32)
                for q in range(_DIM // 16):
                    sl = pl.ds(q * 16, 16)
                    sbuf[eidx, sl] = gbuf[eidx, sl] * wb
            return c2

        lax.fori_loop(0, _K // 16, group, 0)

    for r in range(_R):
        idx_copy(r, r).start()
    idx_copy(0, 0).wait()
    row_gather(0, 0).start()
    idx_copy(1, 1).wait()
    row_gather(1, 1).start()

    def block(i, carry):
        jb = i * _R
        for r in range(_R):
            j = jb + r
            row_gather(r, j).wait()
            @pl.when(i > 0)
            def _drain():
                row_scatter_wait(r)
            for q in range(_K // 16):
                dstr_v[r, pl.ds(q * 16, 16)] = (
                    idx_v[pl.ds(r * _REC + _K + q * 16, 16)])
            scale(r, j)
            @pl.when(j + _R < _CH)
            def _iref():
                idx_copy(r, j + _R).start()
            rr = (r + 2) % _R
            @pl.when(j + 2 < _CH)
            def _gref():
                idx_copy(rr, j + 2).wait()
                row_gather(rr, j + 2).start()
            row_scatter(r)
        return carry

    lax.fori_loop(0, _CH // _R, block, 0)
    for r in range(_R):
        row_scatter_wait(r)
    plsc.subcore_barrier()

    pltpu.sync_copy(agg_sh.at[pl.ds(base, _RPT)], out_hbm.at[c, s])


def kernel(features, edge_index, adj0_weight, adj1_weight,
           kernel_mean, kernel_var, noise):
    x2 = pl.pallas_call(
        _pre_body,
        grid=(_N // _BN,),
        in_specs=[
            pl.BlockSpec((_BN, 2 * _DIM), lambda i: (i, 0)),
            pl.BlockSpec((_DIM, _DIM), lambda i: (0, 0)),
            pl.BlockSpec((_DIM, _DIM), lambda i: (0, 0)),
        ],
        out_specs=pl.BlockSpec((2, _BN, _DIM), lambda i: (0, i, 0)),
        out_shape=jax.ShapeDtypeStruct((2, _N, _DIM), jnp.float32),
    )(features, kernel_mean, kernel_var)
    x2f = x2.reshape(2 * _N, _DIM)

    dst = edge_index[0]
    src = edge_index[1]
    pad = _EPAD - _E
    srcp = jnp.pad(src, (0, pad)).reshape(_NS, _CH, _K)
    dstp = jnp.pad(dst, (0, pad)).reshape(_NS, _CH, _K)
    src2 = srcp[None] + (jnp.arange(_NC, dtype=jnp.int32) * _N)[:, None, None, None]
    dst2 = jnp.broadcast_to(dstp[None], (_NC, _NS, _CH, _K))
    w2 = jnp.stack([
        jnp.pad(adj0_weight, (0, pad)),
        jnp.pad(adj1_weight, (0, pad)),
    ]).reshape(_NC, _NS, _CH, _K)
    wbits = lax.bitcast_convert_type(w2, jnp.int32)
    edata = jnp.concatenate([src2, dst2, wbits], axis=-1)

    agg = _edge_kernel(x2f, edata).reshape(_NC, _N, _DIM)

    out = pl.pallas_call(
        _post_body,
        grid=(_N // _BN,),
        in_specs=[
            pl.BlockSpec((2, _BN, _DIM), lambda i: (0, i, 0)),
            pl.BlockSpec((_BN, _DIM), lambda i: (i, 0)),
        ],
        out_specs=pl.BlockSpec((_BN, _DIM), lambda i: (i, 0)),
        out_shape=jax.ShapeDtypeStruct((_N, _DIM), jnp.float32),
    )(agg, noise)
    return out


# speedup vs baseline: 1.1402x; 1.1402x over previous
import functools

import jax
import jax.numpy as jnp
from jax import lax
from jax.experimental import pallas as pl
from jax.experimental.pallas import tpu as pltpu
from jax.experimental.pallas import tpu_sc as plsc

_N = 10000
_E = 320000
_DIM = 64
_GAMMA = 1.0

_NC = 2
_NS = 16
_K = 128
_CH = 160
_R = 4
_EPT = _CH * _K
_EPAD = _NS * _EPT
_RPT = _N // _NS

_BN = 2000


def _pre_body(f_ref, km_ref, kv_ref, out_ref):
    f = f_ref[...]
    dn = (((1,), (0,)), ((), ()))
    m = lax.dot_general(f[:, :_DIM], km_ref[...], dn,
                        precision=lax.Precision.HIGHEST,
                        preferred_element_type=jnp.float32)
    v = lax.dot_general(f[:, _DIM:], kv_ref[...], dn,
                        precision=lax.Precision.HIGHEST,
                        preferred_element_type=jnp.float32)
    m = jnp.where(m > 0.0, m, jnp.exp(m) - 1.0)
    v = jnp.maximum(v, 0.0)
    att = jnp.exp(-_GAMMA * v)
    out_ref[0] = m * att
    out_ref[1] = v * att * att


def _post_body(agg_ref, noise_ref, out_ref):
    out_ref[...] = agg_ref[0] + jnp.sqrt(agg_ref[1] + 1e-8) * noise_ref[...]


_sc_mesh = plsc.VectorSubcoreMesh(
    core_axis_name="c", subcore_axis_name="s", num_cores=_NC, num_subcores=_NS
)


@functools.partial(
    pl.kernel,
    out_type=jax.ShapeDtypeStruct((_NC, _NS, _RPT, _DIM), jnp.float32),
    mesh=_sc_mesh,
    compiler_params=pltpu.CompilerParams(
        needs_layout_passes=False, use_tc_tiling_on_sc=False),
    scratch_types=(
        [
            pltpu.VMEM((_CH, _K), jnp.int16),
            pltpu.VMEM((_CH, _K), jnp.int16),
            pltpu.VMEM((_EPT,), jnp.float32),
            pltpu.VMEM((_R, _K), jnp.int32),
            pltpu.VMEM((_R, _K), jnp.int32),
        ]
        + [pltpu.VMEM((_K, _DIM), jnp.float32)] * _R
        + [pltpu.VMEM_SHARED((_N, _DIM), jnp.float32)]
        + [pltpu.SemaphoreType.DMA] * (2 * _R)
    ),
)
def _edge_kernel(x2_hbm, src_hbm, dst_hbm, w_hbm, out_hbm,
                 src16_v, dst16_v, w_v, src32_v, dst32_v,
                 g0, g1, g2, g3, agg_sh,
                 q0, q1, q2, q3, p0, p1, p2, p3):
    c = lax.axis_index("c")
    s = lax.axis_index("s")
    gbufs = (g0, g1, g2, g3)
    gsems = (q0, q1, q2, q3)
    ssems = (p0, p1, p2, p3)

    pltpu.sync_copy(src_hbm.at[c, s], src16_v)
    pltpu.sync_copy(dst_hbm.at[s], dst16_v)
    pltpu.sync_copy(w_hbm.at[c, s], w_v)

    def zrow(e, carry):
        for q in range(_DIM // 16):
            g0[e, pl.ds(q * 16, 16)] = jnp.zeros((16,), jnp.float32)
        return carry

    lax.fori_loop(0, _K, zrow, 0)

    base = s * _RPT
    n_full = _RPT // _K
    rem = _RPT - n_full * _K

    def zcp(i, carry):
        pltpu.sync_copy(g0, agg_sh.at[pl.ds(base + i * _K, _K)])
        return carry

    lax.fori_loop(0, n_full, zcp, 0)
    if rem:
        pltpu.sync_copy(g0.at[pl.ds(0, rem)],
                        agg_sh.at[pl.ds(base + n_full * _K, rem)])
    plsc.subcore_barrier()

    def unpack16(ref16, j, ref32, r):
        for q in range(_K // 32):
            ab = ref16[j, pl.ds(q * 32, 32)]
            lo, hi = plsc.unpack(ab, format=plsc.PackFormat.INTERLEAVED)
            ref32[r, pl.ds(q * 32, 16)] = lo
            ref32[r, pl.ds(q * 32 + 16, 16)] = hi

    def gather(r, j):
        return pltpu.make_async_copy(
            x2_hbm.at[src32_v.at[r]], gbufs[r], gsems[r])

    def scatter_desc(r):
        return pltpu.make_async_copy(
            gbufs[r], agg_sh.at[dst32_v.at[r]], ssems[r])

    def scale(r, j):
        gbuf = gbufs[r]
        wbase = j * _K

        def group(g, c2):
            for e in range(16):
                eidx = g * 16 + e
                wb = plsc.load_gather(
                    w_v, [jnp.full((16,), wbase + eidx, jnp.int32)])
                for q in range(_DIM // 16):
                    sl = pl.ds(q * 16, 16)
                    gbuf[eidx, sl] = gbuf[eidx, sl] * wb
            return c2

        lax.fori_loop(0, _K // 16, group, 0)

    unpack16(src16_v, 0, src32_v, 0)
    gather(0, 0).start()
    unpack16(src16_v, 1, src32_v, 1)
    gather(1, 1).start()

    def block(i, carry):
        jb = i * _R
        for r in range(_R):
            j = jb + r
            gather(r, j).wait()
            unpack16(dst16_v, j, dst32_v, r)
            scale(r, j)
            pltpu.async_copy(
                gbufs[r], agg_sh.at[dst32_v.at[r]], ssems[r], add=True)
            rn = (r + 2) % _R

            @pl.when(j + 2 < _CH)
            def _refill():
                if r >= 2:
                    scatter_desc(rn).wait()
                else:
                    @pl.when(i > 0)
                    def _drain():
                        scatter_desc(rn).wait()
                unpack16(src16_v, j + 2, src32_v, rn)
                gather(rn, j + 2).start()
        return carry

    lax.fori_loop(0, _CH // _R, block, 0)
    for r in range(_R):
        scatter_desc(r).wait()
    plsc.subcore_barrier()

    pltpu.sync_copy(agg_sh.at[pl.ds(base, _RPT)], out_hbm.at[c, s])


def _interleave16(x):
    lead = x.shape[:-1]
    y = x.reshape(lead + (_K // 32, 2, 16))
    return jnp.swapaxes(y, -1, -2).reshape(lead + (_K,))


def kernel(features, edge_index, adj0_weight, adj1_weight,
           kernel_mean, kernel_var, noise):
    x2 = pl.pallas_call(
        _pre_body,
        grid=(_N // _BN,),
        in_specs=[
            pl.BlockSpec((_BN, 2 * _DIM), lambda i: (i, 0)),
            pl.BlockSpec((_DIM, _DIM), lambda i: (0, 0)),
            pl.BlockSpec((_DIM, _DIM), lambda i: (0, 0)),
        ],
        out_specs=pl.BlockSpec((2, _BN, _DIM), lambda i: (0, i, 0)),
        out_shape=jax.ShapeDtypeStruct((2, _N, _DIM), jnp.float32),
    )(features, kernel_mean, kernel_var)
    x2f = x2.reshape(2 * _N, _DIM)

    dst = edge_index[0]
    src = edge_index[1]
    pad = _EPAD - _E
    srcp = jnp.pad(src, (0, pad)).reshape(_NS, _CH, _K)
    dstp = jnp.pad(dst, (0, pad)).reshape(_NS, _CH, _K)
    src2 = srcp[None] + (jnp.arange(_NC, dtype=jnp.int32) * _N)[:, None, None, None]
    src16 = _interleave16(src2.astype(jnp.int16))
    dst16 = _interleave16(dstp.astype(jnp.int16))
    w2 = jnp.stack([
        jnp.pad(adj0_weight, (0, pad)),
        jnp.pad(adj1_weight, (0, pad)),
    ]).reshape(_NC, _NS, _EPT)

    agg = _edge_kernel(x2f, src16, dst16, w2).reshape(_NC, _N, _DIM)

    out = pl.pallas_call(
        _post_body,
        grid=(_N // _BN,),
        in_specs=[
            pl.BlockSpec((2, _BN, _DIM), lambda i: (0, i, 0)),
            pl.BlockSpec((_BN, _DIM), lambda i: (i, 0)),
        ],
        out_specs=pl.BlockSpec((_BN, _DIM), lambda i: (i, 0)),
        out_shape=jax.ShapeDtypeStruct((_N, _DIM), jnp.float32),
    )(agg, noise)
    return out
